# baseline (speedup 1.0000x reference)
import functools

import jax
import jax.numpy as jnp
from jax import lax
from jax.experimental import pallas as pl
from jax.experimental.pallas import tpu as pltpu
from jax.experimental.pallas import tpu_sc as plsc


_NUM_CORES = 2
_NUM_SUBCORES = 16


def kernel(x, pos_embedding):
    batch, seq_len, head_dim = x.shape
    nw = _NUM_CORES * _NUM_SUBCORES
    rows_per_w = seq_len // nw
    mesh = plsc.VectorSubcoreMesh(core_axis_name="c", subcore_axis_name="s")

    @functools.partial(
        pl.kernel,
        mesh=mesh,
        out_type=jax.ShapeDtypeStruct(
            (batch, seq_len, head_dim), pos_embedding.dtype
        ),
        scratch_types=[
            pltpu.VMEM((rows_per_w, head_dim), pos_embedding.dtype),
            pltpu.SemaphoreType.DMA,
        ],
    )
    def _sc_copy(table_hbm, out_hbm, rows_v, sem):
        wid = lax.axis_index("s") * _NUM_CORES + lax.axis_index("c")
        base = wid * rows_per_w
        pltpu.sync_copy(table_hbm.at[pl.ds(base, rows_per_w)], rows_v)
        copies = [
            pltpu.make_async_copy(
                rows_v, out_hbm.at[b, pl.ds(base, rows_per_w)], sem
            )
            for b in range(batch)
        ]
        for c in copies:
            c.start()
        for c in copies:
            c.wait()

    return _sc_copy(pos_embedding)


# speedup vs baseline: 1.0159x; 1.0159x over previous
import functools

import jax
import jax.numpy as jnp
from jax import lax
from jax.experimental import pallas as pl
from jax.experimental.pallas import tpu as pltpu
from jax.experimental.pallas import tpu_sc as plsc


_NUM_CORES = 2
_NUM_SUBCORES = 16


def kernel(x, pos_embedding):
    batch, seq_len, head_dim = x.shape
    nw = _NUM_CORES * _NUM_SUBCORES
    rows_per_w = seq_len // nw
    mesh = plsc.VectorSubcoreMesh(core_axis_name="c", subcore_axis_name="s")

    @functools.partial(
        pl.kernel,
        mesh=mesh,
        out_type=jax.ShapeDtypeStruct(
            (batch, seq_len, head_dim), pos_embedding.dtype
        ),
        scratch_types=[
            pltpu.VMEM((rows_per_w, head_dim), pos_embedding.dtype),
            pltpu.SemaphoreType.DMA,
            pltpu.SemaphoreType.DMA,
        ],
    )
    def _sc_copy(table_hbm, out_hbm, rows_v, in_sem, out_sem):
        wid = lax.axis_index("s") * _NUM_CORES + lax.axis_index("c")
        base = wid * rows_per_w
        n_sub = 4
        sub = rows_per_w // n_sub
        in_copies = [
            pltpu.make_async_copy(
                table_hbm.at[pl.ds(base + i * sub, sub)],
                rows_v.at[pl.ds(i * sub, sub)],
                in_sem,
            )
            for i in range(n_sub)
        ]
        for c in in_copies:
            c.start()
        out_copies = []
        for i in range(n_sub):
            in_copies[i].wait()
            for b in range(batch):
                c = pltpu.make_async_copy(
                    rows_v.at[pl.ds(i * sub, sub)],
                    out_hbm.at[b, pl.ds(base + i * sub, sub)],
                    out_sem,
                )
                c.start()
                out_copies.append(c)
        for c in out_copies:
            c.wait()

    return _sc_copy(pos_embedding)


# speedup vs baseline: 2.3698x; 2.3327x over previous
import jax
import jax.numpy as jnp
from jax.experimental import pallas as pl
from jax.experimental.pallas import tpu as pltpu


_N_CHUNKS = 32


def _make_dma_kernel(batch, seq_len, head_dim):
    ch = seq_len // _N_CHUNKS

    def _dma_kernel(pos_ref, out_ref, vmem, in_sems, out_sems):
        for i in range(_N_CHUNKS):
            pltpu.make_async_copy(
                pos_ref.at[pl.ds(i * ch, ch)],
                vmem.at[pl.ds(i * ch, ch)],
                in_sems.at[i],
            ).start()
        for i in range(_N_CHUNKS):
            pltpu.make_async_copy(
                pos_ref.at[pl.ds(i * ch, ch)],
                vmem.at[pl.ds(i * ch, ch)],
                in_sems.at[i],
            ).wait()
            for b in range(batch):
                pltpu.make_async_copy(
                    vmem.at[pl.ds(i * ch, ch)],
                    out_ref.at[b, pl.ds(i * ch, ch)],
                    out_sems.at[b],
                ).start()
        for i in range(_N_CHUNKS):
            for b in range(batch):
                pltpu.make_async_copy(
                    vmem.at[pl.ds(i * ch, ch)],
                    out_ref.at[b, pl.ds(i * ch, ch)],
                    out_sems.at[b],
                ).wait()

    return _dma_kernel


def kernel(x, pos_embedding):
    batch, seq_len, head_dim = x.shape
    return pl.pallas_call(
        _make_dma_kernel(batch, seq_len, head_dim),
        in_specs=[pl.BlockSpec(memory_space=pl.ANY)],
        out_specs=pl.BlockSpec(memory_space=pl.ANY),
        out_shape=jax.ShapeDtypeStruct(
            (batch, seq_len, head_dim), pos_embedding.dtype
        ),
        scratch_shapes=[
            pltpu.VMEM((seq_len, head_dim), pos_embedding.dtype),
            pltpu.SemaphoreType.DMA((_N_CHUNKS,)),
            pltpu.SemaphoreType.DMA((batch,)),
        ],
    )(pos_embedding)


# speedup vs baseline: 2.4132x; 1.0183x over previous
import jax
import jax.numpy as jnp
from jax.experimental import pallas as pl
from jax.experimental.pallas import tpu as pltpu


_N_CHUNKS = 16


def _make_dma_kernel(batch, seq_len, head_dim):
    ch = seq_len // _N_CHUNKS

    def _dma_kernel(pos_ref, out_ref, vmem, in_sems, out_sems):
        for i in range(_N_CHUNKS):
            pltpu.make_async_copy(
                pos_ref.at[pl.ds(i * ch, ch)],
                vmem.at[pl.ds(i * ch, ch)],
                in_sems.at[i],
            ).start()
        for i in range(_N_CHUNKS):
            pltpu.make_async_copy(
                pos_ref.at[pl.ds(i * ch, ch)],
                vmem.at[pl.ds(i * ch, ch)],
                in_sems.at[i],
            ).wait()
            for b in range(batch):
                pltpu.make_async_copy(
                    vmem.at[pl.ds(i * ch, ch)],
                    out_ref.at[b, pl.ds(i * ch, ch)],
                    out_sems.at[b],
                ).start()
        for i in range(_N_CHUNKS):
            for b in range(batch):
                pltpu.make_async_copy(
                    vmem.at[pl.ds(i * ch, ch)],
                    out_ref.at[b, pl.ds(i * ch, ch)],
                    out_sems.at[b],
                ).wait()

    return _dma_kernel


def kernel(x, pos_embedding):
    batch, seq_len, head_dim = x.shape
    return pl.pallas_call(
        _make_dma_kernel(batch, seq_len, head_dim),
        in_specs=[pl.BlockSpec(memory_space=pl.ANY)],
        out_specs=pl.BlockSpec(memory_space=pl.ANY),
        out_shape=jax.ShapeDtypeStruct(
            (batch, seq_len, head_dim), pos_embedding.dtype
        ),
        scratch_shapes=[
            pltpu.VMEM((seq_len, head_dim), pos_embedding.dtype),
            pltpu.SemaphoreType.DMA((_N_CHUNKS,)),
            pltpu.SemaphoreType.DMA((batch,)),
        ],
    )(pos_embedding)
